# baseline (speedup 1.0000x reference)
import jax
import jax.numpy as jnp
from jax.experimental import pallas as pl

DEPTH = 1000
BLOCK = 512


def _onehot_body(idx_ref, out_ref):
    idx = idx_ref[0, 0, :]
    iota = jax.lax.broadcasted_iota(jnp.int32, (BLOCK, DEPTH), 1)
    out_ref[...] = (idx[:, None] == iota).astype(jnp.float32)


def kernel(inputs):
    b, s = inputs.shape
    n = b * s
    nb = n // BLOCK
    idx = inputs.astype(jnp.int32).reshape(nb, 1, BLOCK)
    out = pl.pallas_call(
        _onehot_body,
        grid=(nb,),
        in_specs=[pl.BlockSpec((1, 1, BLOCK), lambda i: (i, 0, 0))],
        out_specs=pl.BlockSpec((BLOCK, DEPTH), lambda i: (i, 0)),
        out_shape=jax.ShapeDtypeStruct((n, DEPTH), jnp.float32),
    )(idx)
    return out.reshape(b, s, DEPTH)
